# baseline (speedup 1.0000x reference)
import functools

import jax
import jax.numpy as jnp
from jax import lax
from jax.experimental import pallas as pl
from jax.experimental.pallas import tpu as pltpu
from jax.experimental.pallas import tpu_sc as plsc

_B, _S = 16384, 200
_D = 64
_V = 500
_VP = 512
_NC, _NS = 2, 16
_NW = _NC * _NS
_BS = _B // _NW
_L = 16
_NG = _BS // _L


def _lookup(table_t, x_t):
    mesh = plsc.VectorSubcoreMesh(core_axis_name="c", subcore_axis_name="s")

    @functools.partial(
        pl.kernel,
        mesh=mesh,
        out_type=jax.ShapeDtypeStruct((_S, _D, _B), jnp.float32),
        compiler_params=pltpu.CompilerParams(
            use_tc_tiling_on_sc=True, needs_layout_passes=False),
        scratch_types=[
            pltpu.VMEM((_D * _VP,), jnp.float32),
            pltpu.VMEM((2, _BS), jnp.int32),
            pltpu.VMEM((2, _D, _BS), jnp.float32),
            pltpu.SemaphoreType.DMA,
            pltpu.SemaphoreType.DMA,
            pltpu.SemaphoreType.DMA,
            pltpu.SemaphoreType.DMA,
        ],
    )
    def k(tab_hbm, xt_hbm, out_hbm, tab_v, idx_v, blk_v, isem0, isem1,
          osem0, osem1):
        wid = lax.axis_index("s") * _NC + lax.axis_index("c")
        bs = wid * _BS

        pltpu.sync_copy(tab_hbm, tab_v)

        def idx_load(l, h):
            isem = isem0 if h == 0 else isem1
            return pltpu.async_copy(
                xt_hbm.at[pl.ds(l * _B + bs, _BS)], idx_v.at[h], isem)

        def compute_block(h):
            @plsc.parallel_loop(0, _NG, step=1, unroll=2)
            def per_group(g):
                idx_vec = idx_v[h, pl.ds(g * _L, _L)]
                vals = [
                    plsc.load_gather(tab_v, [idx_vec + t * _VP])
                    for t in range(8)
                ]
                for n in range(1, 8):
                    nxt = [
                        plsc.load_gather(tab_v, [idx_vec + (n * 8 + t) * _VP])
                        for t in range(8)
                    ]
                    for t in range(8):
                        blk_v[h, (n - 1) * 8 + t, pl.ds(g * _L, _L)] = vals[t]
                    vals = nxt
                for t in range(8):
                    blk_v[h, 56 + t, pl.ds(g * _L, _L)] = vals[t]

        def write_block(l, h):
            osem = osem0 if h == 0 else osem1
            return pltpu.async_copy(
                blk_v.at[h], out_hbm.at[l, :, pl.ds(bs, _BS)], osem)

        def drain_write(h):
            osem = osem0 if h == 0 else osem1
            pltpu.make_async_copy(
                blk_v.at[h], out_hbm.at[0, :, pl.ds(bs, _BS)], osem).wait()

        def drain_idx(h):
            isem = isem0 if h == 0 else isem1
            pltpu.make_async_copy(
                xt_hbm.at[pl.ds(bs, _BS)], idx_v.at[h], isem).wait()

        idx_load(0, 0)
        idx_load(1, 1)

        def pair(p, carry):
            l0 = 2 * p
            for h in (0, 1):
                l = l0 + h
                drain_idx(h)

                @pl.when(l >= 2)
                def _():
                    drain_write(h)

                compute_block(h)
                write_block(l, h)

                @pl.when(l + 2 < _S)
                def _():
                    idx_load(l + 2, h)
            return carry

        lax.fori_loop(0, _S // 2, pair, 0)
        drain_write(0)
        drain_write(1)

    return k(table_t, x_t)


def kernel(x, pe_table):
    x_t = x.T.reshape(_S * _B)
    table_t = jnp.pad(pe_table.T, ((0, 0), (0, _VP - _V))).reshape(_D * _VP)
    out_phys = _lookup(table_t, x_t)
    return jnp.transpose(out_phys, (2, 0, 1))


# speedup vs baseline: 2.4941x; 2.4941x over previous
import functools

import jax
import jax.numpy as jnp
from jax import lax
from jax.experimental import pallas as pl
from jax.experimental.pallas import tpu as pltpu
from jax.experimental.pallas import tpu_sc as plsc

_B, _S = 16384, 200
_D = 64
_DH = _D // 2
_V = 500
_VP = 512
_NC, _NS = 2, 16
_NW = _NC * _NS
_BS = _B // _NW
_L = 16
_NG = _BS // _L
_LAG = 6


def _lookup(table_t, x_t):
    mesh = plsc.VectorSubcoreMesh(core_axis_name="c", subcore_axis_name="s")

    @functools.partial(
        pl.kernel,
        mesh=mesh,
        out_type=jax.ShapeDtypeStruct((_S, _D, _B), jnp.float32),
        compiler_params=pltpu.CompilerParams(
            use_tc_tiling_on_sc=True, needs_layout_passes=False),
        scratch_types=[
            pltpu.VMEM((_D * _VP,), jnp.float32),
            pltpu.VMEM((2, _BS), jnp.int32),
            pltpu.VMEM((4, _DH, _BS), jnp.float32),
            pltpu.SemaphoreType.DMA,
            pltpu.SemaphoreType.DMA,
            pltpu.SemaphoreType.DMA,
            pltpu.SemaphoreType.DMA,
            pltpu.SemaphoreType.DMA,
            pltpu.SemaphoreType.DMA,
        ],
    )
    def k(tab_hbm, xt_hbm, out_hbm, tab_v, idx_v, blk_v, isem0, isem1,
          osem0, osem1, osem2, osem3):
        wid = lax.axis_index("s") * _NC + lax.axis_index("c")
        bs = wid * _BS
        isems = (isem0, isem1)
        osems = (osem0, osem1, osem2, osem3)

        pltpu.sync_copy(tab_hbm, tab_v)

        def idx_load(l, h):
            return pltpu.async_copy(
                xt_hbm.at[pl.ds(l * _B + bs, _BS)], idx_v.at[h], isems[h])

        def compute_half(h, half, slot):
            def per_group(g, carry):
                idx_vec = idx_v[h, pl.ds(g * _L, _L)]
                pend = {}
                for t in range(_DH + _LAG):
                    if t < _DH:
                        d = half * _DH + t
                        pend[t] = plsc.load_gather(
                            tab_v.at[pl.ds(d * _VP, _VP)], [idx_vec])
                    if t >= _LAG:
                        blk_v[slot, t - _LAG, pl.ds(g * _L, _L)] = \
                            pend.pop(t - _LAG)
                return carry
            lax.fori_loop(0, _NG, per_group, 0)

        def write_half(l, half, slot):
            return pltpu.async_copy(
                blk_v.at[slot],
                out_hbm.at[l, pl.ds(half * _DH, _DH), pl.ds(bs, _BS)],
                osems[slot])

        def drain_write(slot):
            pltpu.make_async_copy(
                blk_v.at[slot],
                out_hbm.at[0, pl.ds(0, _DH), pl.ds(bs, _BS)],
                osems[slot]).wait()

        def drain_idx(h):
            pltpu.make_async_copy(
                xt_hbm.at[pl.ds(bs, _BS)], idx_v.at[h], isems[h]).wait()

        idx_load(0, 0)
        idx_load(1, 1)

        def pair(p, carry):
            l0 = 2 * p
            for h in (0, 1):
                l = l0 + h
                drain_idx(h)
                for half in (0, 1):
                    slot = 2 * h + half

                    @pl.when(p >= 1)
                    def _():
                        drain_write(slot)

                    compute_half(h, half, slot)
                    write_half(l, half, slot)

                @pl.when(l + 2 < _S)
                def _():
                    idx_load(l + 2, h)
            return carry

        lax.fori_loop(0, _S // 2, pair, 0)
        for slot in range(4):
            drain_write(slot)

    return k(table_t, x_t)


def kernel(x, pe_table):
    x_t = x.T.reshape(_S * _B)
    table_t = jnp.pad(pe_table.T, ((0, 0), (0, _VP - _V))).reshape(_D * _VP)
    out_phys = _lookup(table_t, x_t)
    return jnp.transpose(out_phys, (2, 0, 1))


# speedup vs baseline: 2.5769x; 1.0332x over previous
import functools

import jax
import jax.numpy as jnp
from jax import lax
from jax.experimental import pallas as pl
from jax.experimental.pallas import tpu as pltpu
from jax.experimental.pallas import tpu_sc as plsc

_B, _S = 16384, 200
_D = 64
_V = 500
_VP = 512
_NC, _NS = 2, 16
_NW = _NC * _NS
_BS = _B // _NW
_L = 16
_NG = _BS // _L


def _lookup(table_t, x_t):
    mesh = plsc.VectorSubcoreMesh(core_axis_name="c", subcore_axis_name="s")

    @functools.partial(
        pl.kernel,
        mesh=mesh,
        out_type=jax.ShapeDtypeStruct((_S, _D, _B), jnp.float32),
        compiler_params=pltpu.CompilerParams(
            use_tc_tiling_on_sc=True, needs_layout_passes=False),
        scratch_types=[
            pltpu.VMEM((_D * _VP,), jnp.float32),
            pltpu.VMEM((2, _BS), jnp.int32),
            pltpu.VMEM((2, _D, _BS), jnp.float32),
            pltpu.SemaphoreType.DMA,
            pltpu.SemaphoreType.DMA,
            pltpu.SemaphoreType.DMA,
            pltpu.SemaphoreType.DMA,
        ],
    )
    def k(tab_hbm, xt_hbm, out_hbm, tab_v, idx_v, blk_v, isem0, isem1,
          osem0, osem1):
        wid = lax.axis_index("s") * _NC + lax.axis_index("c")
        bs = wid * _BS

        pltpu.sync_copy(tab_hbm, tab_v)

        def idx_load(l, h):
            isem = isem0 if h == 0 else isem1
            return pltpu.async_copy(
                xt_hbm.at[pl.ds(l * _B + bs, _BS)], idx_v.at[h], isem)

        def compute_block(h):
            def per_group(g, carry):
                idx_vec = idx_v[h, pl.ds(g * _L, _L)]
                pend = {}
                for t in range(_D + 6):
                    if t < _D:
                        pend[t] = plsc.load_gather(
                            tab_v.at[pl.ds(t * _VP, _VP)], [idx_vec])
                    if t >= 6:
                        blk_v[h, t - 6, pl.ds(g * _L, _L)] = pend.pop(t - 6)
                return carry
            lax.fori_loop(0, _NG, per_group, 0)

        def write_block(l, h):
            osem = osem0 if h == 0 else osem1
            return pltpu.async_copy(
                blk_v.at[h], out_hbm.at[l, :, pl.ds(bs, _BS)], osem)

        def drain_write(h):
            osem = osem0 if h == 0 else osem1
            pltpu.make_async_copy(
                blk_v.at[h], out_hbm.at[0, :, pl.ds(bs, _BS)], osem).wait()

        def drain_idx(h):
            isem = isem0 if h == 0 else isem1
            pltpu.make_async_copy(
                xt_hbm.at[pl.ds(bs, _BS)], idx_v.at[h], isem).wait()

        idx_load(0, 0)
        idx_load(1, 1)

        def pair(p, carry):
            l0 = 2 * p
            for h in (0, 1):
                l = l0 + h
                drain_idx(h)

                @pl.when(l >= 2)
                def _():
                    drain_write(h)

                compute_block(h)
                write_block(l, h)

                @pl.when(l + 2 < _S)
                def _():
                    idx_load(l + 2, h)
            return carry

        lax.fori_loop(0, _S // 2, pair, 0)
        drain_write(0)
        drain_write(1)

    return k(table_t, x_t)


def kernel(x, pe_table):
    x_t = x.T.reshape(_S * _B)
    table_t = jnp.pad(pe_table.T, ((0, 0), (0, _VP - _V))).reshape(_D * _VP)
    out_phys = _lookup(table_t, x_t)
    return jnp.transpose(out_phys, (2, 0, 1))
